# baseline (speedup 1.0000x reference)
import dataclasses
import functools
import math

import jax
import jax.numpy as jnp
from jax import lax
from jax.experimental import pallas as pl
from jax.experimental.pallas import tpu as pltpu
from jax.experimental.pallas import tpu_sc as plsc

NC = 2
NS = 16
NW = NC * NS
L = 16
K = 64
SB = 14


def _sc_aggregate(node_emb, src, dst):
    Bb, Nn, Dd = node_emb.shape
    Ee = src.shape[0]
    npt = (Nn // NW) // 8 * 8
    npt_last = Nn - (NW - 1) * npt
    trash = npt_last
    acc_rows = (npt_last + 1 + 7) // 8 * 8
    ech = 1600
    n_ech = Ee // ech
    lw = 128
    lrows = 60
    cap = lrows * lw - 2 * K

    mesh = plsc.VectorSubcoreMesh(core_axis_name="c", subcore_axis_name="s")
    cparams = pltpu.CompilerParams()
    if "needs_layout_passes" in pltpu.CompilerParams.__dataclass_fields__:
        cparams = dataclasses.replace(cparams, needs_layout_passes=False)

    @functools.partial(
        pl.kernel,
        out_type=jax.ShapeDtypeStruct((Bb, Nn, Dd), jnp.float32),
        mesh=mesh,
        compiler_params=cparams,
        scratch_types=[
            pltpu.VMEM((ech,), jnp.int32),
            pltpu.VMEM((ech,), jnp.int32),
            pltpu.VMEM((lrows, lw), jnp.int32),
            pltpu.VMEM((K,), jnp.int32),
            pltpu.VMEM((K,), jnp.int32),
            pltpu.VMEM((K, Dd), jnp.float32),
            pltpu.VMEM((K, Dd), jnp.float32),
            pltpu.VMEM((acc_rows, Dd), jnp.float32),
            pltpu.SemaphoreType.DMA,
            pltpu.SemaphoreType.DMA,
        ],
    )
    def agg_kernel(emb_hbm, src_hbm, dst_hbm, out_hbm,
                   sbuf, dbuf, plist, sidx0, sidx1, rowbuf0, rowbuf1, acc,
                   sem0, sem1):
        c = lax.axis_index("c")
        s = lax.axis_index("s")
        w = s * NC + c
        start = w * npt
        my_rows = jnp.where(w == NW - 1, npt_last, npt)

        iota = lax.iota(jnp.int32, L)
        zeros_f = jnp.zeros((L,), jnp.float32)
        trash_v = jnp.full((L,), trash << SB, jnp.int32)

        def filt_chunk(cnt, nedge):
            def filt_body(i, cnt):
                sv = sbuf[pl.ds(i * L, L)]
                dv = dbuf[pl.ds(i * L, L)]
                localv = dv - start
                mask = (localv >= 0) & (localv < my_rows)
                mi = jnp.where(mask, 1, 0).astype(jnp.int32)
                pos = jnp.full((L,), cnt, jnp.int32) + plsc.cumsum(mi) - 1
                row = pos // lw
                col = pos - row * lw
                packed = sv | (localv << SB)
                plsc.store_scatter(plist, [row, col], packed, mask=mask)
                return cnt + jnp.sum(mi)
            return plsc.parallel_loop(0, nedge // L, carry=cnt, unroll=2)(filt_body)

        def pad_tail(cnt):
            for t in range(2 * K // L):
                pos = jnp.full((L,), cnt + t * L, jnp.int32) + iota
                row = pos // lw
                col = pos - row * lw
                plsc.store_scatter(plist, [row, col], trash_v)

        rvecs = [jnp.full((L,), q * L, jnp.int32) + iota for q in range(K // L)]
        lvec = iota * 17

        def unpack_chunk(j, sidx_ref):
            jr = j // (lw // K)
            jo = (j - jr * (lw // K)) * K
            for q in range(K // L):
                p = plist[jr, pl.ds(jo + q * L, L)]
                sidx_ref[pl.ds(q * L, L)] = p & ((1 << SB) - 1)

        def gather_dma(b, sidx_ref, rowbuf_ref, sem):
            return pltpu.make_async_copy(
                emb_hbm.at[b].at[sidx_ref], rowbuf_ref, sem)

        def accumulate(j, rowbuf_ref):
            jr = j // (lw // K)
            jo = (j - jr * (lw // K)) * K
            dvecs = []
            for q in range(K // L):
                p = plist[jr, pl.ds(jo + q * L, L)]
                dvecs.append(p >> SB)

            @plsc.parallel_loop(0, Dd, unroll=4)
            def _(cc):
                colv = (jnp.full((L,), cc, jnp.int32) + lvec) & (Dd - 1)
                for q in range(K // L):
                    x = plsc.load_gather(rowbuf_ref, [rvecs[q], colv])
                    plsc.addupdate_scatter(acc, [dvecs[q], colv], x)

        def process_chunk(b, j):
            unpack_chunk(j, sidx0)
            pltpu.sync_copy(emb_hbm.at[b].at[sidx0], rowbuf0)
            accumulate(j, rowbuf0)

        def zero_acc():
            def z_body(r, _):
                for q in range(Dd // L):
                    acc[r, pl.ds(q * L, L)] = zeros_f
                return 0
            lax.fori_loop(0, acc_rows, z_body, 0)

        def copy_out(b):
            @pl.when(w < NW - 1)
            def _():
                pltpu.sync_copy(acc.at[pl.ds(0, npt)],
                                out_hbm.at[b].at[pl.ds(start, npt)])

            @pl.when(w == NW - 1)
            def _():
                pltpu.sync_copy(acc.at[pl.ds(0, npt_last)],
                                out_hbm.at[b].at[pl.ds((NW - 1) * npt, npt_last)])

        def scan_chunk(ci, cnt):
            pltpu.sync_copy(src_hbm.at[pl.ds(ci * ech, ech)], sbuf)
            pltpu.sync_copy(dst_hbm.at[pl.ds(ci * ech, ech)], dbuf)
            return filt_chunk(cnt, ech)

        cnt = lax.fori_loop(0, n_ech, scan_chunk, jnp.int32(0))

        @pl.when(cnt <= cap)
        def _fast():
            pad_tail(cnt)
            nch = (cnt + K - 1) // K
            nch2 = (nch + 1) // 2 * 2
            npairs = nch2 // 2
            for b in range(Bb):
                zero_acc()

                @pl.when(npairs > 0)
                def _():
                    unpack_chunk(0, sidx0)
                    gather_dma(b, sidx0, rowbuf0, sem0).start()

                    def pair_body(i, _):
                        j0 = 2 * i
                        unpack_chunk(j0 + 1, sidx1)
                        gather_dma(b, sidx0, rowbuf0, sem0).wait()
                        gather_dma(b, sidx1, rowbuf1, sem1).start()
                        accumulate(j0, rowbuf0)

                        @pl.when(j0 + 2 < nch2)
                        def _():
                            unpack_chunk(j0 + 2, sidx0)
                            gather_dma(b, sidx0, rowbuf0, sem0).start()

                        gather_dma(b, sidx1, rowbuf1, sem1).wait()
                        accumulate(j0 + 1, rowbuf1)
                        return 0
                    lax.fori_loop(0, npairs, pair_body, 0)
                copy_out(b)

        @pl.when(cnt > cap)
        def _slow():
            for b in range(Bb):
                zero_acc()

                def sc_body(ci, _):
                    pltpu.sync_copy(src_hbm.at[pl.ds(ci * ech, ech)], sbuf)
                    pltpu.sync_copy(dst_hbm.at[pl.ds(ci * ech, ech)], dbuf)
                    cnt_c = filt_chunk(jnp.int32(0), ech)
                    pad_tail(cnt_c)
                    nch_c = (cnt_c + K - 1) // K

                    def chunk_body(j, _):
                        process_chunk(b, j)
                        return 0
                    lax.fori_loop(0, nch_c, chunk_body, 0)
                    return 0
                lax.fori_loop(0, n_ech, sc_body, 0)
                copy_out(b)

    return agg_kernel(node_emb, src, dst)


def _tc_dense(agg, node_emb, W, bvec, gamma, beta):
    Bb, Nn, Dd = agg.shape
    BN = 1000
    grid = (Bb, Nn // BN)
    inv_sqrt2 = 1.0 / math.sqrt(2.0)

    def body(agg_ref, emb_ref, w_ref, b_ref, g_ref, bt_ref, out_ref):
        x = agg_ref[0]
        y = jnp.dot(x, w_ref[...], preferred_element_type=jnp.float32,
                    precision=lax.Precision.HIGHEST)
        y = y + b_ref[0]
        mean = jnp.mean(y, axis=1, keepdims=True)
        yc = y - mean
        var = jnp.mean(yc * yc, axis=1, keepdims=True)
        y = yc * lax.rsqrt(var + 1e-5) * g_ref[0] + bt_ref[0]
        y = 0.5 * y * (1.0 + lax.erf(y * inv_sqrt2))
        out_ref[0] = y + emb_ref[0]

    return pl.pallas_call(
        body,
        grid=grid,
        in_specs=[
            pl.BlockSpec((1, BN, Dd), lambda b, n: (b, n, 0)),
            pl.BlockSpec((1, BN, Dd), lambda b, n: (b, n, 0)),
            pl.BlockSpec((Dd, Dd), lambda b, n: (0, 0)),
            pl.BlockSpec((1, Dd), lambda b, n: (0, 0)),
            pl.BlockSpec((1, Dd), lambda b, n: (0, 0)),
            pl.BlockSpec((1, Dd), lambda b, n: (0, 0)),
        ],
        out_specs=pl.BlockSpec((1, BN, Dd), lambda b, n: (b, n, 0)),
        out_shape=jax.ShapeDtypeStruct((Bb, Nn, Dd), jnp.float32),
    )(agg, node_emb, W, bvec, gamma, beta)


def kernel(node_embeddings, edges, W, b, gamma, beta):
    src = jnp.asarray(edges[:, 0], jnp.int32)
    dst = jnp.asarray(edges[:, 1], jnp.int32)
    agg = _sc_aggregate(node_embeddings, src, dst)
    return _tc_dense(agg, node_embeddings, W,
                     b.reshape(1, -1), gamma.reshape(1, -1), beta.reshape(1, -1))


# speedup vs baseline: 1.0852x; 1.0852x over previous
import dataclasses
import functools
import math

import jax
import jax.numpy as jnp
from jax import lax
from jax.experimental import pallas as pl
from jax.experimental.pallas import tpu as pltpu
from jax.experimental.pallas import tpu_sc as plsc

NC = 2
NS = 16
NW = NC * NS
L = 16
K = 64
SB = 14


def _sc_aggregate(node_emb, src, dst):
    Bb, Nn, Dd = node_emb.shape
    Ee = src.shape[0]
    npt = (Nn // NW) // 8 * 8
    npt_last = Nn - (NW - 1) * npt
    trash = npt_last
    acc_rows = (npt_last + 1 + 7) // 8 * 8
    ech = 800
    n_ech = Ee // ech
    lw = 128
    lrows = 60
    cap = lrows * lw - 2 * K

    mesh = plsc.VectorSubcoreMesh(core_axis_name="c", subcore_axis_name="s")
    cparams = pltpu.CompilerParams()
    if "needs_layout_passes" in pltpu.CompilerParams.__dataclass_fields__:
        cparams = dataclasses.replace(cparams, needs_layout_passes=False)

    @functools.partial(
        pl.kernel,
        out_type=jax.ShapeDtypeStruct((Bb, Nn, Dd), jnp.float32),
        mesh=mesh,
        compiler_params=cparams,
        scratch_types=[
            pltpu.VMEM((ech,), jnp.int32),
            pltpu.VMEM((ech,), jnp.int32),
            pltpu.VMEM((ech,), jnp.int32),
            pltpu.VMEM((ech,), jnp.int32),
            pltpu.VMEM((lrows, lw), jnp.int32),
            pltpu.VMEM((K,), jnp.int32),
            pltpu.VMEM((K,), jnp.int32),
            pltpu.VMEM((K, Dd), jnp.float32),
            pltpu.VMEM((K, Dd), jnp.float32),
            pltpu.VMEM((acc_rows, Dd), jnp.float32),
            pltpu.SemaphoreType.DMA,
            pltpu.SemaphoreType.DMA,
        ],
    )
    def agg_kernel(emb_hbm, src_hbm, dst_hbm, out_hbm,
                   sbuf0, dbuf0, sbuf1, dbuf1, plist, sidx0, sidx1,
                   rowbuf0, rowbuf1, acc, sem0, sem1):
        c = lax.axis_index("c")
        s = lax.axis_index("s")
        w = s * NC + c
        start = w * npt
        my_rows = jnp.where(w == NW - 1, npt_last, npt)

        iota = lax.iota(jnp.int32, L)
        zeros_f = jnp.zeros((L,), jnp.float32)
        trash_v = jnp.full((L,), trash << SB, jnp.int32)

        def filt_chunk(cnt, nedge, sbuf, dbuf):
            def filt_body(i, cnt):
                sv = sbuf[pl.ds(i * L, L)]
                dv = dbuf[pl.ds(i * L, L)]
                localv = dv - start
                mask = (localv >= 0) & (localv < my_rows)
                mi = jnp.where(mask, 1, 0).astype(jnp.int32)
                pos = jnp.full((L,), cnt, jnp.int32) + plsc.cumsum(mi) - 1
                row = pos // lw
                col = pos - row * lw
                packed = sv | (localv << SB)
                plsc.store_scatter(plist, [row, col], packed, mask=mask)
                return cnt + jnp.sum(mi)
            return plsc.parallel_loop(0, nedge // L, carry=cnt, unroll=2)(filt_body)

        def pad_tail(cnt):
            for t in range(2 * K // L):
                pos = jnp.full((L,), cnt + t * L, jnp.int32) + iota
                row = pos // lw
                col = pos - row * lw
                plsc.store_scatter(plist, [row, col], trash_v)

        rvecs = [jnp.full((L,), q * L, jnp.int32) + iota for q in range(K // L)]
        lvec = iota * 17

        def unpack_chunk(j, sidx_ref):
            jr = j // (lw // K)
            jo = (j - jr * (lw // K)) * K
            for q in range(K // L):
                p = plist[jr, pl.ds(jo + q * L, L)]
                sidx_ref[pl.ds(q * L, L)] = p & ((1 << SB) - 1)

        def gather_dma(b, sidx_ref, rowbuf_ref, sem):
            return pltpu.make_async_copy(
                emb_hbm.at[b].at[sidx_ref], rowbuf_ref, sem)

        def accumulate(j, rowbuf_ref):
            jr = j // (lw // K)
            jo = (j - jr * (lw // K)) * K
            dvecs = []
            for q in range(K // L):
                p = plist[jr, pl.ds(jo + q * L, L)]
                dvecs.append(p >> SB)

            @plsc.parallel_loop(0, Dd, unroll=4)
            def _(cc):
                colv = (jnp.full((L,), cc, jnp.int32) + lvec) & (Dd - 1)
                for q in range(K // L):
                    x = plsc.load_gather(rowbuf_ref, [rvecs[q], colv])
                    plsc.addupdate_scatter(acc, [dvecs[q], colv], x)

        def process_chunk(b, j):
            unpack_chunk(j, sidx0)
            pltpu.sync_copy(emb_hbm.at[b].at[sidx0], rowbuf0)
            accumulate(j, rowbuf0)

        def zero_acc():
            def z_body(r, _):
                for q in range(Dd // L):
                    acc[r, pl.ds(q * L, L)] = zeros_f
                return 0
            lax.fori_loop(0, acc_rows, z_body, 0)

        def copy_out(b):
            @pl.when(w < NW - 1)
            def _():
                pltpu.sync_copy(acc.at[pl.ds(0, npt)],
                                out_hbm.at[b].at[pl.ds(start, npt)])

            @pl.when(w == NW - 1)
            def _():
                pltpu.sync_copy(acc.at[pl.ds(0, npt_last)],
                                out_hbm.at[b].at[pl.ds((NW - 1) * npt, npt_last)])

        def stage_dma(ci, sbuf_ref, dbuf_ref, sem):
            return (pltpu.make_async_copy(src_hbm.at[pl.ds(ci * ech, ech)],
                                          sbuf_ref, sem),
                    pltpu.make_async_copy(dst_hbm.at[pl.ds(ci * ech, ech)],
                                          dbuf_ref, sem))

        def issue_stage(ci, sbuf_ref, dbuf_ref, sem):
            a, d = stage_dma(ci, sbuf_ref, dbuf_ref, sem)
            a.start()
            d.start()

        def wait_stage(ci, sbuf_ref, dbuf_ref, sem):
            a, d = stage_dma(ci, sbuf_ref, dbuf_ref, sem)
            a.wait()
            d.wait()

        issue_stage(0, sbuf0, dbuf0, sem0)

        def scan_pair(i, cnt):
            c0 = 2 * i
            issue_stage(c0 + 1, sbuf1, dbuf1, sem1)
            wait_stage(c0, sbuf0, dbuf0, sem0)
            cnt = filt_chunk(cnt, ech, sbuf0, dbuf0)

            @pl.when(c0 + 2 < n_ech)
            def _():
                issue_stage(c0 + 2, sbuf0, dbuf0, sem0)

            wait_stage(c0 + 1, sbuf1, dbuf1, sem1)
            return filt_chunk(cnt, ech, sbuf1, dbuf1)

        cnt = lax.fori_loop(0, n_ech // 2, scan_pair, jnp.int32(0))

        @pl.when(cnt <= cap)
        def _fast():
            pad_tail(cnt)
            nch = (cnt + K - 1) // K
            nch2 = (nch + 1) // 2 * 2
            npairs = nch2 // 2
            for b in range(Bb):
                zero_acc()

                @pl.when(npairs > 0)
                def _():
                    unpack_chunk(0, sidx0)
                    gather_dma(b, sidx0, rowbuf0, sem0).start()

                    def pair_body(i, _):
                        j0 = 2 * i
                        unpack_chunk(j0 + 1, sidx1)
                        gather_dma(b, sidx0, rowbuf0, sem0).wait()
                        gather_dma(b, sidx1, rowbuf1, sem1).start()
                        accumulate(j0, rowbuf0)

                        @pl.when(j0 + 2 < nch2)
                        def _():
                            unpack_chunk(j0 + 2, sidx0)
                            gather_dma(b, sidx0, rowbuf0, sem0).start()

                        gather_dma(b, sidx1, rowbuf1, sem1).wait()
                        accumulate(j0 + 1, rowbuf1)
                        return 0
                    lax.fori_loop(0, npairs, pair_body, 0)
                copy_out(b)

        @pl.when(cnt > cap)
        def _slow():
            for b in range(Bb):
                zero_acc()

                def sc_body(ci, _):
                    pltpu.sync_copy(src_hbm.at[pl.ds(ci * ech, ech)], sbuf0)
                    pltpu.sync_copy(dst_hbm.at[pl.ds(ci * ech, ech)], dbuf0)
                    cnt_c = filt_chunk(jnp.int32(0), ech, sbuf0, dbuf0)
                    pad_tail(cnt_c)
                    nch_c = (cnt_c + K - 1) // K

                    def chunk_body(j, _):
                        process_chunk(b, j)
                        return 0
                    lax.fori_loop(0, nch_c, chunk_body, 0)
                    return 0
                lax.fori_loop(0, n_ech, sc_body, 0)
                copy_out(b)

    return agg_kernel(node_emb, src, dst)


def _tc_dense(agg, node_emb, W, bvec, gamma, beta):
    Bb, Nn, Dd = agg.shape
    BN = 1000
    grid = (Bb, Nn // BN)
    inv_sqrt2 = 1.0 / math.sqrt(2.0)

    def body(agg_ref, emb_ref, w_ref, b_ref, g_ref, bt_ref, out_ref):
        x = agg_ref[0]
        y = jnp.dot(x, w_ref[...], preferred_element_type=jnp.float32,
                    precision=lax.Precision.HIGHEST)
        y = y + b_ref[0]
        mean = jnp.mean(y, axis=1, keepdims=True)
        yc = y - mean
        var = jnp.mean(yc * yc, axis=1, keepdims=True)
        y = yc * lax.rsqrt(var + 1e-5) * g_ref[0] + bt_ref[0]
        y = 0.5 * y * (1.0 + lax.erf(y * inv_sqrt2))
        out_ref[0] = y + emb_ref[0]

    return pl.pallas_call(
        body,
        grid=grid,
        in_specs=[
            pl.BlockSpec((1, BN, Dd), lambda b, n: (b, n, 0)),
            pl.BlockSpec((1, BN, Dd), lambda b, n: (b, n, 0)),
            pl.BlockSpec((Dd, Dd), lambda b, n: (0, 0)),
            pl.BlockSpec((1, Dd), lambda b, n: (0, 0)),
            pl.BlockSpec((1, Dd), lambda b, n: (0, 0)),
            pl.BlockSpec((1, Dd), lambda b, n: (0, 0)),
        ],
        out_specs=pl.BlockSpec((1, BN, Dd), lambda b, n: (b, n, 0)),
        out_shape=jax.ShapeDtypeStruct((Bb, Nn, Dd), jnp.float32),
    )(agg, node_emb, W, bvec, gamma, beta)


def kernel(node_embeddings, edges, W, b, gamma, beta):
    src = jnp.asarray(edges[:, 0], jnp.int32)
    dst = jnp.asarray(edges[:, 1], jnp.int32)
    agg = _sc_aggregate(node_embeddings, src, dst)
    return _tc_dense(agg, node_embeddings, W,
                     b.reshape(1, -1), gamma.reshape(1, -1), beta.reshape(1, -1))
